# baseline (speedup 1.0000x reference)
import functools

import jax
import jax.numpy as jnp
from jax import lax
from jax.experimental import pallas as pl
from jax.experimental.pallas import tpu as pltpu
from jax.experimental.pallas import tpu_sc as plsc

N = 10000
D = 128
NC = 2
NS = 16
NW = NC * NS
K = 128
CH = 80
E_PAD = NW * CH * K
N_SP = 10112
RPT = N_SP // NS
PAD_ROW = N
CH_H = CH // 2

_mesh = plsc.VectorSubcoreMesh(core_axis_name="c", subcore_axis_name="s")


def _deg_body(dstm_hbm, out_hbm, dst_v, rows_v, acc, sem):
    cid = lax.axis_index("c")
    sid = lax.axis_index("s")
    wid = cid * NS + sid

    @pl.loop(0, K)
    def _(r):
        @pl.loop(0, D, step=16)
        def _(c):
            rows_v[r, pl.ds(c, 16)] = jnp.zeros((16,), jnp.float32)

    base = sid * RPT
    @pl.loop(0, RPT - (RPT % K), step=K)
    def _(j):
        pltpu.sync_copy(rows_v, acc.at[pl.ds(base + j, K)])
    pltpu.sync_copy(rows_v.at[pl.ds(0, RPT % K)],
                    acc.at[pl.ds(base + RPT - (RPT % K), RPT % K)])

    @pl.loop(0, K)
    def _(r):
        rows_v[r, pl.ds(0, 16)] = jnp.ones((16,), jnp.float32)
    plsc.subcore_barrier()

    pltpu.sync_copy(dstm_hbm.at[pl.ds(wid * CH, CH)], dst_v)

    @pl.loop(0, CH)
    def _(j):
        pltpu.sync_copy(rows_v, acc.at[dst_v.at[j]], add=True)

    plsc.subcore_barrier()
    pltpu.sync_copy(acc.at[pl.ds(base, RPT)], out_hbm.at[cid, pl.ds(base, RPT)])


@jax.jit
def _sc_deg(dstm):
    f = pl.kernel(
        _deg_body,
        out_type=jax.ShapeDtypeStruct((NC, N_SP, D), jnp.float32),
        mesh=_mesh,
        scratch_types=[
            pltpu.VMEM((CH, K), jnp.int32),
            pltpu.VMEM((K, D), jnp.float32),
            pltpu.VMEM_SHARED((N_SP, D), jnp.float32),
            pltpu.SemaphoreType.DMA,
        ],
    )
    return f(dstm)


def _scatter_body(ga_hbm, gb_hbm, srcm_hbm, dstm_hbm, out_hbm, src_v, dst_v,
                  rows_a, rows_b, acc, sem_ga, sem_gb, sem_sa, sem_sb):
    cid = lax.axis_index("c")
    sid = lax.axis_index("s")
    wid = cid * NS + sid

    @pl.loop(0, K)
    def _(r):
        @pl.loop(0, D, step=16)
        def _(c):
            rows_a[r, pl.ds(c, 16)] = jnp.zeros((16,), jnp.float32)

    base = sid * RPT
    @pl.loop(0, RPT - (RPT % K), step=K)
    def _(j):
        pltpu.sync_copy(rows_a, acc.at[pl.ds(base + j, K)])
    pltpu.sync_copy(rows_a.at[pl.ds(0, RPT % K)],
                    acc.at[pl.ds(base + RPT - (RPT % K), RPT % K)])
    plsc.subcore_barrier()

    pltpu.sync_copy(dstm_hbm.at[pl.ds(wid * CH, CH)], dst_v)

    def run(g_hbm):
        for h in range(2):
            pltpu.sync_copy(srcm_hbm.at[pl.ds(wid * CH + h * CH_H, CH_H)],
                            src_v)
            pltpu.async_copy(g_hbm.at[src_v.at[0]], rows_a, sem_ga)
            pltpu.async_copy(g_hbm.at[src_v.at[1]], rows_b, sem_gb)

            @pl.loop(0, CH_H, step=2)
            def _(j):
                dj = h * CH_H + j
                pltpu.make_async_copy(g_hbm.at[src_v.at[j]], rows_a,
                                      sem_ga).wait()
                pltpu.async_copy(rows_a, acc.at[dst_v.at[dj]], sem_sa,
                                 add=True)
                pltpu.make_async_copy(g_hbm.at[src_v.at[j + 1]], rows_b,
                                      sem_gb).wait()
                pltpu.async_copy(rows_b, acc.at[dst_v.at[dj + 1]], sem_sb,
                                 add=True)
                pltpu.make_async_copy(rows_a, acc.at[dst_v.at[dj]],
                                      sem_sa).wait()

                @pl.when(j + 2 < CH_H)
                def _():
                    pltpu.async_copy(g_hbm.at[src_v.at[j + 2]], rows_a, sem_ga)
                pltpu.make_async_copy(rows_b, acc.at[dst_v.at[dj + 1]],
                                      sem_sb).wait()

                @pl.when(j + 3 < CH_H)
                def _():
                    pltpu.async_copy(g_hbm.at[src_v.at[j + 3]], rows_b, sem_gb)

    @pl.when(cid == 0)
    def _():
        run(ga_hbm)

    @pl.when(cid == 1)
    def _():
        run(gb_hbm)

    plsc.subcore_barrier()
    pltpu.sync_copy(acc.at[pl.ds(base, RPT)], out_hbm.at[cid, pl.ds(base, RPT)])


@jax.jit
def _sc_scatter(ga, gb, srcm, dstm):
    f = pl.kernel(
        _scatter_body,
        out_type=jax.ShapeDtypeStruct((NC, N_SP, D), jnp.float32),
        mesh=_mesh,
        scratch_types=[
            pltpu.VMEM((CH_H, K), jnp.int32),
            pltpu.VMEM((CH, K), jnp.int32),
            pltpu.VMEM((K, D), jnp.float32),
            pltpu.VMEM((K, D), jnp.float32),
            pltpu.VMEM_SHARED((N_SP, D), jnp.float32),
            pltpu.SemaphoreType.DMA,
            pltpu.SemaphoreType.DMA,
            pltpu.SemaphoreType.DMA,
            pltpu.SemaphoreType.DMA,
        ],
    )
    return f(ga, gb, srcm, dstm)


def _tc1_body(x_ref, w1_ref, degp_ref, g1_ref, g1b_ref, dis_ref):
    deg = degp_ref[0, :N, 0:1] + degp_ref[1, :N, 0:1] + 1.0
    dis = lax.rsqrt(deg)
    h = jnp.dot(x_ref[...], w1_ref[...], preferred_element_type=jnp.float32)
    g1 = h * dis
    g1_ref[...] = g1
    g1b_ref[...] = g1
    dis_ref[...] = dis


def _tc2_body(g1_ref, aggp_ref, dis_ref, w2_ref, b1_ref, g2_ref, g2b_ref):
    agg = aggp_ref[0, :N, :] + aggp_ref[1, :N, :]
    dis = dis_ref[...]
    s = (agg + g1_ref[...]) * dis + b1_ref[...]
    h2 = jnp.maximum(s, 0.0)
    g2 = jnp.dot(h2, w2_ref[...], preferred_element_type=jnp.float32) * dis
    g2_ref[...] = g2
    g2b_ref[...] = g2


def _tc3_body(g2_ref, aggp_ref, dis_ref, b2_ref, ge_ref, out_ref):
    agg = aggp_ref[0, :N, :] + aggp_ref[1, :N, :]
    out = (agg + g2_ref[...]) * dis_ref[...] + b2_ref[...]
    out_ref[...] = out
    ge_ref[...] = jnp.mean(out, axis=0, keepdims=True)


@jax.jit
def _tc1(x, W1, degp):
    return pl.pallas_call(
        _tc1_body,
        out_shape=[jax.ShapeDtypeStruct((N, D), jnp.float32),
                   jax.ShapeDtypeStruct((N, D), jnp.float32),
                   jax.ShapeDtypeStruct((N, 1), jnp.float32)],
    )(x, W1, degp)


@jax.jit
def _tc2(g1, aggp, dis, W2, b1):
    return pl.pallas_call(
        _tc2_body,
        out_shape=[jax.ShapeDtypeStruct((N, D), jnp.float32),
                   jax.ShapeDtypeStruct((N, D), jnp.float32)],
    )(g1, aggp, dis, W2, b1)


@jax.jit
def _tc3(g2, aggp, dis, b2):
    return pl.pallas_call(
        _tc3_body,
        out_shape=[jax.ShapeDtypeStruct((1, D), jnp.float32),
                   jax.ShapeDtypeStruct((N, D), jnp.float32)],
    )(g2, aggp, dis, b2)


@jax.jit
def kernel(x, edge_index, W1, b1, W2, b2):
    ei = edge_index.astype(jnp.int32)
    pad = E_PAD - ei.shape[1]
    srcm = jnp.concatenate(
        [ei[0], jnp.zeros((pad,), jnp.int32)]).reshape(NW * CH, K)
    dstm = jnp.concatenate(
        [ei[1], jnp.full((pad,), PAD_ROW, jnp.int32)]).reshape(NW * CH, K)

    degp = _sc_deg(dstm)
    g1, g1b, dis = _tc1(x, W1, degp)
    aggp1 = _sc_scatter(g1, g1b, srcm, dstm)
    g2, g2b = _tc2(g1, aggp1, dis, W2, b1.reshape(1, D))
    aggp2 = _sc_scatter(g2, g2b, srcm, dstm)
    ge, out = _tc3(g2, aggp2, dis, b2.reshape(1, D))
    return (ge, out)


# speedup vs baseline: 1.1039x; 1.1039x over previous
import functools

import jax
import jax.numpy as jnp
from jax import lax
from jax.experimental import pallas as pl
from jax.experimental.pallas import tpu as pltpu
from jax.experimental.pallas import tpu_sc as plsc

N = 10000
D = 128
NC = 2
NS = 16
NW = NC * NS
K = 128
CH = 80
CH0 = 128
CH1 = 32
CH0_H = CH0 // 2
CH1_H = CH1 // 2
E_PAD = NW * CH * K
N_SP = 10112
RPT = N_SP // NS
PAD_ROW = N
CH_H = CH // 2

_mesh = plsc.VectorSubcoreMesh(core_axis_name="c", subcore_axis_name="s")


def _deg_body(dstm_hbm, out_hbm, dst_v, rows_v, acc, sem):
    cid = lax.axis_index("c")
    sid = lax.axis_index("s")
    wid = cid * NS + sid

    @pl.loop(0, K)
    def _(r):
        @pl.loop(0, D, step=16)
        def _(c):
            rows_v[r, pl.ds(c, 16)] = jnp.zeros((16,), jnp.float32)

    base = sid * RPT
    @pl.loop(0, RPT - (RPT % K), step=K)
    def _(j):
        pltpu.sync_copy(rows_v, acc.at[pl.ds(base + j, K)])
    pltpu.sync_copy(rows_v.at[pl.ds(0, RPT % K)],
                    acc.at[pl.ds(base + RPT - (RPT % K), RPT % K)])

    @pl.loop(0, K)
    def _(r):
        rows_v[r, pl.ds(0, 16)] = jnp.ones((16,), jnp.float32)
    plsc.subcore_barrier()

    pltpu.sync_copy(dstm_hbm.at[pl.ds(wid * CH, CH)], dst_v)

    @pl.loop(0, CH)
    def _(j):
        pltpu.sync_copy(rows_v, acc.at[dst_v.at[j]], add=True)

    plsc.subcore_barrier()
    pltpu.sync_copy(acc.at[pl.ds(base, RPT)], out_hbm.at[cid, pl.ds(base, RPT)])


@jax.jit
def _sc_deg(dstm):
    f = pl.kernel(
        _deg_body,
        out_type=jax.ShapeDtypeStruct((NC, N_SP, D), jnp.float32),
        mesh=_mesh,
        scratch_types=[
            pltpu.VMEM((CH, K), jnp.int32),
            pltpu.VMEM((K, D), jnp.float32),
            pltpu.VMEM_SHARED((N_SP, D), jnp.float32),
            pltpu.SemaphoreType.DMA,
        ],
    )
    return f(dstm)


def _scatter_body(ga_hbm, gb_hbm, srcm_hbm, dstm_hbm, out_hbm, src_v, dst_v,
                  rows_a, rows_b, acc, sem_ga, sem_gb, sem_sa, sem_sb):
    cid = lax.axis_index("c")
    sid = lax.axis_index("s")
    wid = cid * NS + sid

    @pl.loop(0, K)
    def _(r):
        @pl.loop(0, D, step=16)
        def _(c):
            rows_a[r, pl.ds(c, 16)] = jnp.zeros((16,), jnp.float32)

    base = sid * RPT
    @pl.loop(0, RPT - (RPT % K), step=K)
    def _(j):
        pltpu.sync_copy(rows_a, acc.at[pl.ds(base + j, K)])
    pltpu.sync_copy(rows_a.at[pl.ds(0, RPT % K)],
                    acc.at[pl.ds(base + RPT - (RPT % K), RPT % K)])
    plsc.subcore_barrier()

    def run(g_hbm, ch_half, rowbase):
        for h in range(2):
            rb = rowbase + h * ch_half
            pltpu.sync_copy(srcm_hbm.at[pl.ds(rb, ch_half)],
                            src_v.at[pl.ds(0, ch_half)])
            pltpu.sync_copy(dstm_hbm.at[pl.ds(rb, ch_half)],
                            dst_v.at[pl.ds(0, ch_half)])
            pltpu.async_copy(g_hbm.at[src_v.at[0]], rows_a, sem_ga)
            pltpu.async_copy(g_hbm.at[src_v.at[1]], rows_b, sem_gb)

            @pl.loop(0, ch_half, step=2)
            def _(j):
                pltpu.make_async_copy(g_hbm.at[src_v.at[j]], rows_a,
                                      sem_ga).wait()
                pltpu.async_copy(rows_a, acc.at[dst_v.at[j]], sem_sa,
                                 add=True)
                pltpu.make_async_copy(g_hbm.at[src_v.at[j + 1]], rows_b,
                                      sem_gb).wait()
                pltpu.async_copy(rows_b, acc.at[dst_v.at[j + 1]], sem_sb,
                                 add=True)
                pltpu.make_async_copy(rows_a, acc.at[dst_v.at[j]],
                                      sem_sa).wait()

                @pl.when(j + 2 < ch_half)
                def _():
                    pltpu.async_copy(g_hbm.at[src_v.at[j + 2]], rows_a, sem_ga)
                pltpu.make_async_copy(rows_b, acc.at[dst_v.at[j + 1]],
                                      sem_sb).wait()

                @pl.when(j + 3 < ch_half)
                def _():
                    pltpu.async_copy(g_hbm.at[src_v.at[j + 3]], rows_b, sem_gb)

    @pl.when(cid == 0)
    def _():
        run(ga_hbm, CH0_H, sid * CH0)

    @pl.when(cid == 1)
    def _():
        run(gb_hbm, CH1_H, NS * CH0 + sid * CH1)

    plsc.subcore_barrier()
    pltpu.sync_copy(acc.at[pl.ds(base, RPT)], out_hbm.at[cid, pl.ds(base, RPT)])


@jax.jit
def _sc_scatter(ga, gb, srcm, dstm):
    f = pl.kernel(
        _scatter_body,
        out_type=jax.ShapeDtypeStruct((NC, N_SP, D), jnp.float32),
        mesh=_mesh,
        scratch_types=[
            pltpu.VMEM((CH0_H, K), jnp.int32),
            pltpu.VMEM((CH0_H, K), jnp.int32),
            pltpu.VMEM((K, D), jnp.float32),
            pltpu.VMEM((K, D), jnp.float32),
            pltpu.VMEM_SHARED((N_SP, D), jnp.float32),
            pltpu.SemaphoreType.DMA,
            pltpu.SemaphoreType.DMA,
            pltpu.SemaphoreType.DMA,
            pltpu.SemaphoreType.DMA,
        ],
    )
    return f(ga, gb, srcm, dstm)


def _tc1_body(x_ref, w1_ref, degp_ref, g1_ref, g1b_ref, dis_ref):
    deg = degp_ref[0, :N, 0:1] + degp_ref[1, :N, 0:1] + 1.0
    dis = lax.rsqrt(deg)
    h = jnp.dot(x_ref[...], w1_ref[...], preferred_element_type=jnp.float32)
    g1 = h * dis
    g1_ref[...] = g1
    g1b_ref[...] = g1
    dis_ref[...] = dis


def _tc2_body(g1_ref, aggp_ref, dis_ref, w2_ref, b1_ref, g2_ref, g2b_ref):
    agg = aggp_ref[0, :N, :] + aggp_ref[1, :N, :]
    dis = dis_ref[...]
    s = (agg + g1_ref[...]) * dis + b1_ref[...]
    h2 = jnp.maximum(s, 0.0)
    g2 = jnp.dot(h2, w2_ref[...], preferred_element_type=jnp.float32) * dis
    g2_ref[...] = g2
    g2b_ref[...] = g2


def _tc3_body(g2_ref, aggp_ref, dis_ref, b2_ref, ge_ref, out_ref):
    agg = aggp_ref[0, :N, :] + aggp_ref[1, :N, :]
    out = (agg + g2_ref[...]) * dis_ref[...] + b2_ref[...]
    out_ref[...] = out
    ge_ref[...] = jnp.mean(out, axis=0, keepdims=True)


@jax.jit
def _tc1(x, W1, degp):
    return pl.pallas_call(
        _tc1_body,
        out_shape=[jax.ShapeDtypeStruct((N, D), jnp.float32),
                   jax.ShapeDtypeStruct((N, D), jnp.float32),
                   jax.ShapeDtypeStruct((N, 1), jnp.float32)],
    )(x, W1, degp)


@jax.jit
def _tc2(g1, aggp, dis, W2, b1):
    return pl.pallas_call(
        _tc2_body,
        out_shape=[jax.ShapeDtypeStruct((N, D), jnp.float32),
                   jax.ShapeDtypeStruct((N, D), jnp.float32)],
    )(g1, aggp, dis, W2, b1)


@jax.jit
def _tc3(g2, aggp, dis, b2):
    return pl.pallas_call(
        _tc3_body,
        out_shape=[jax.ShapeDtypeStruct((1, D), jnp.float32),
                   jax.ShapeDtypeStruct((N, D), jnp.float32)],
    )(g2, aggp, dis, b2)


@jax.jit
def kernel(x, edge_index, W1, b1, W2, b2):
    ei = edge_index.astype(jnp.int32)
    pad = E_PAD - ei.shape[1]
    srcm = jnp.concatenate(
        [ei[0], jnp.zeros((pad,), jnp.int32)]).reshape(NW * CH, K)
    dstm = jnp.concatenate(
        [ei[1], jnp.full((pad,), PAD_ROW, jnp.int32)]).reshape(NW * CH, K)

    degp = _sc_deg(dstm)
    g1, g1b, dis = _tc1(x, W1, degp)
    aggp1 = _sc_scatter(g1, g1b, srcm, dstm)
    g2, g2b = _tc2(g1, aggp1, dis, W2, b1.reshape(1, D))
    aggp2 = _sc_scatter(g2, g2b, srcm, dstm)
    ge, out = _tc3(g2, aggp2, dis, b2.reshape(1, D))
    return (ge, out)


# speedup vs baseline: 1.3112x; 1.1878x over previous
import functools

import jax
import jax.numpy as jnp
from jax import lax
from jax.experimental import pallas as pl
from jax.experimental.pallas import tpu as pltpu
from jax.experimental.pallas import tpu_sc as plsc

N = 10000
D = 128
NC = 2
NS = 16
NW = NC * NS
K = 128
CH = 80
CH0 = 144
CH1 = 16
ST0 = 24
ST1 = 8
E_PAD = NW * CH * K
N_SP = 10112
RPT = N_SP // NS
PAD_ROW = N
CH_H = CH // 2

_mesh = plsc.VectorSubcoreMesh(core_axis_name="c", subcore_axis_name="s")


def _deg_body(dstm_hbm, out_hbm, dst_v, rows_v, acc, sem):
    cid = lax.axis_index("c")
    sid = lax.axis_index("s")
    wid = cid * NS + sid

    @pl.loop(0, K)
    def _(r):
        @pl.loop(0, D, step=16)
        def _(c):
            rows_v[r, pl.ds(c, 16)] = jnp.zeros((16,), jnp.float32)

    base = sid * RPT
    @pl.loop(0, RPT - (RPT % K), step=K)
    def _(j):
        pltpu.sync_copy(rows_v, acc.at[pl.ds(base + j, K)])
    pltpu.sync_copy(rows_v.at[pl.ds(0, RPT % K)],
                    acc.at[pl.ds(base + RPT - (RPT % K), RPT % K)])

    @pl.loop(0, K)
    def _(r):
        rows_v[r, pl.ds(0, 16)] = jnp.ones((16,), jnp.float32)
    plsc.subcore_barrier()

    pltpu.sync_copy(dstm_hbm.at[pl.ds(wid * CH, CH)], dst_v)

    @pl.loop(0, CH)
    def _(j):
        pltpu.sync_copy(rows_v, acc.at[dst_v.at[j]], add=True)

    plsc.subcore_barrier()
    pltpu.sync_copy(acc.at[pl.ds(base, RPT)], out_hbm.at[cid, pl.ds(base, RPT)])


@jax.jit
def _sc_deg(dstm):
    f = pl.kernel(
        _deg_body,
        out_type=jax.ShapeDtypeStruct((NC, N_SP, D), jnp.float32),
        mesh=_mesh,
        scratch_types=[
            pltpu.VMEM((CH, K), jnp.int32),
            pltpu.VMEM((K, D), jnp.float32),
            pltpu.VMEM_SHARED((N_SP, D), jnp.float32),
            pltpu.SemaphoreType.DMA,
        ],
    )
    return f(dstm)


def _scatter_body(ga_hbm, gb_hbm, srcm_hbm, dstm_hbm, out_hbm, src_v, dst_v,
                  rows_a, rows_b, acc, sem_ga, sem_gb, sem_sa, sem_sb):
    cid = lax.axis_index("c")
    sid = lax.axis_index("s")
    wid = cid * NS + sid

    @pl.loop(0, K)
    def _(r):
        @pl.loop(0, D, step=16)
        def _(c):
            rows_a[r, pl.ds(c, 16)] = jnp.zeros((16,), jnp.float32)

    base = sid * RPT
    @pl.loop(0, RPT - (RPT % K), step=K)
    def _(j):
        pltpu.sync_copy(rows_a, acc.at[pl.ds(base + j, K)])
    pltpu.sync_copy(rows_a.at[pl.ds(0, RPT % K)],
                    acc.at[pl.ds(base + RPT - (RPT % K), RPT % K)])
    plsc.subcore_barrier()

    def run(g_hbm, ch_half, n_stages, rowbase):
        for h in range(n_stages):
            rb = rowbase + h * ch_half
            pltpu.sync_copy(srcm_hbm.at[pl.ds(rb, ch_half)],
                            src_v.at[pl.ds(0, ch_half)])
            pltpu.sync_copy(dstm_hbm.at[pl.ds(rb, ch_half)],
                            dst_v.at[pl.ds(0, ch_half)])
            pltpu.async_copy(g_hbm.at[src_v.at[0]], rows_a, sem_ga)
            pltpu.async_copy(g_hbm.at[src_v.at[1]], rows_b, sem_gb)

            @pl.loop(0, ch_half, step=2)
            def _(j):
                pltpu.make_async_copy(g_hbm.at[src_v.at[j]], rows_a,
                                      sem_ga).wait()
                pltpu.async_copy(rows_a, acc.at[dst_v.at[j]], sem_sa,
                                 add=True)
                pltpu.make_async_copy(g_hbm.at[src_v.at[j + 1]], rows_b,
                                      sem_gb).wait()
                pltpu.async_copy(rows_b, acc.at[dst_v.at[j + 1]], sem_sb,
                                 add=True)
                pltpu.make_async_copy(rows_a, acc.at[dst_v.at[j]],
                                      sem_sa).wait()

                @pl.when(j + 2 < ch_half)
                def _():
                    pltpu.async_copy(g_hbm.at[src_v.at[j + 2]], rows_a, sem_ga)
                pltpu.make_async_copy(rows_b, acc.at[dst_v.at[j + 1]],
                                      sem_sb).wait()

                @pl.when(j + 3 < ch_half)
                def _():
                    pltpu.async_copy(g_hbm.at[src_v.at[j + 3]], rows_b, sem_gb)

    @pl.when(cid == 0)
    def _():
        run(ga_hbm, ST0, CH0 // ST0, sid * CH0)

    @pl.when(cid == 1)
    def _():
        run(gb_hbm, ST1, CH1 // ST1, NS * CH0 + sid * CH1)

    plsc.subcore_barrier()
    pltpu.sync_copy(acc.at[pl.ds(base, RPT)], out_hbm.at[cid, pl.ds(base, RPT)])


@jax.jit
def _sc_scatter(ga, gb, srcm, dstm):
    f = pl.kernel(
        _scatter_body,
        out_type=jax.ShapeDtypeStruct((NC, N_SP, D), jnp.float32),
        mesh=_mesh,
        scratch_types=[
            pltpu.VMEM((ST0, K), jnp.int32),
            pltpu.VMEM((ST0, K), jnp.int32),
            pltpu.VMEM((K, D), jnp.float32),
            pltpu.VMEM((K, D), jnp.float32),
            pltpu.VMEM_SHARED((N_SP, D), jnp.float32),
            pltpu.SemaphoreType.DMA,
            pltpu.SemaphoreType.DMA,
            pltpu.SemaphoreType.DMA,
            pltpu.SemaphoreType.DMA,
        ],
    )
    return f(ga, gb, srcm, dstm)


def _tc1_body(x_ref, w1_ref, degp_ref, g1_ref, g1b_ref, dis_ref):
    deg = degp_ref[0, :N, 0:1] + degp_ref[1, :N, 0:1] + 1.0
    dis = lax.rsqrt(deg)
    h = jnp.dot(x_ref[...], w1_ref[...], preferred_element_type=jnp.float32)
    g1 = h * dis
    g1_ref[...] = g1
    g1b_ref[...] = g1
    dis_ref[...] = dis


def _tc2_body(g1_ref, aggp_ref, dis_ref, w2_ref, b1_ref, g2_ref, g2b_ref):
    agg = aggp_ref[0, :N, :] + aggp_ref[1, :N, :]
    dis = dis_ref[...]
    s = (agg + g1_ref[...]) * dis + b1_ref[...]
    h2 = jnp.maximum(s, 0.0)
    g2 = jnp.dot(h2, w2_ref[...], preferred_element_type=jnp.float32) * dis
    g2_ref[...] = g2
    g2b_ref[...] = g2


def _tc3_body(g2_ref, aggp_ref, dis_ref, b2_ref, ge_ref, out_ref):
    agg = aggp_ref[0, :N, :] + aggp_ref[1, :N, :]
    out = (agg + g2_ref[...]) * dis_ref[...] + b2_ref[...]
    out_ref[...] = out
    ge_ref[...] = jnp.mean(out, axis=0, keepdims=True)


@jax.jit
def _tc1(x, W1, degp):
    return pl.pallas_call(
        _tc1_body,
        out_shape=[jax.ShapeDtypeStruct((N, D), jnp.float32),
                   jax.ShapeDtypeStruct((N, D), jnp.float32),
                   jax.ShapeDtypeStruct((N, 1), jnp.float32)],
    )(x, W1, degp)


@jax.jit
def _tc2(g1, aggp, dis, W2, b1):
    return pl.pallas_call(
        _tc2_body,
        out_shape=[jax.ShapeDtypeStruct((N, D), jnp.float32),
                   jax.ShapeDtypeStruct((N, D), jnp.float32)],
    )(g1, aggp, dis, W2, b1)


@jax.jit
def _tc3(g2, aggp, dis, b2):
    return pl.pallas_call(
        _tc3_body,
        out_shape=[jax.ShapeDtypeStruct((1, D), jnp.float32),
                   jax.ShapeDtypeStruct((N, D), jnp.float32)],
    )(g2, aggp, dis, b2)


@jax.jit
def kernel(x, edge_index, W1, b1, W2, b2):
    ei = edge_index.astype(jnp.int32)
    pad = E_PAD - ei.shape[1]
    srcm = jnp.concatenate(
        [ei[0], jnp.zeros((pad,), jnp.int32)]).reshape(NW * CH, K)
    dstm = jnp.concatenate(
        [ei[1], jnp.full((pad,), PAD_ROW, jnp.int32)]).reshape(NW * CH, K)

    degp = _sc_deg(dstm)
    g1, g1b, dis = _tc1(x, W1, degp)
    aggp1 = _sc_scatter(g1, g1b, srcm, dstm)
    g2, g2b = _tc2(g1, aggp1, dis, W2, b1.reshape(1, D))
    aggp2 = _sc_scatter(g2, g2b, srcm, dstm)
    ge, out = _tc3(g2, aggp2, dis, b2.reshape(1, D))
    return (ge, out)


# speedup vs baseline: 1.3121x; 1.0007x over previous
import functools

import jax
import jax.numpy as jnp
from jax import lax
from jax.experimental import pallas as pl
from jax.experimental.pallas import tpu as pltpu
from jax.experimental.pallas import tpu_sc as plsc

N = 10000
D = 128
NC = 2
NS = 16
NW = NC * NS
K = 128
CH = 80
CH0 = 144
CH1 = 16
ST0 = 48
ST1 = 16
E_PAD = NW * CH * K
N_SP = 10112
RPT = N_SP // NS
PAD_ROW = N
CH_H = CH // 2

_mesh = plsc.VectorSubcoreMesh(core_axis_name="c", subcore_axis_name="s")


def _deg_body(dstm_hbm, out_hbm, dst_v, rows_v, acc, sem):
    cid = lax.axis_index("c")
    sid = lax.axis_index("s")
    wid = cid * NS + sid

    @pl.loop(0, K)
    def _(r):
        @pl.loop(0, D, step=16)
        def _(c):
            rows_v[r, pl.ds(c, 16)] = jnp.zeros((16,), jnp.float32)

    base = sid * RPT
    @pl.loop(0, RPT - (RPT % K), step=K)
    def _(j):
        pltpu.sync_copy(rows_v, acc.at[pl.ds(base + j, K)])
    pltpu.sync_copy(rows_v.at[pl.ds(0, RPT % K)],
                    acc.at[pl.ds(base + RPT - (RPT % K), RPT % K)])

    @pl.loop(0, K)
    def _(r):
        rows_v[r, pl.ds(0, 16)] = jnp.ones((16,), jnp.float32)
    plsc.subcore_barrier()

    pltpu.sync_copy(dstm_hbm.at[pl.ds(wid * CH, CH)], dst_v)

    @pl.loop(0, CH)
    def _(j):
        pltpu.sync_copy(rows_v, acc.at[dst_v.at[j]], add=True)

    plsc.subcore_barrier()
    pltpu.sync_copy(acc.at[pl.ds(base, RPT)], out_hbm.at[cid, pl.ds(base, RPT)])


@jax.jit
def _sc_deg(dstm):
    f = pl.kernel(
        _deg_body,
        out_type=jax.ShapeDtypeStruct((NC, N_SP, D), jnp.float32),
        mesh=_mesh,
        scratch_types=[
            pltpu.VMEM((CH, K), jnp.int32),
            pltpu.VMEM((K, D), jnp.float32),
            pltpu.VMEM_SHARED((N_SP, D), jnp.float32),
            pltpu.SemaphoreType.DMA,
        ],
    )
    return f(dstm)


def _scatter_body(ga_hbm, gb_hbm, srcm_hbm, dstm_hbm, out_hbm, src_v, dst_v,
                  rows_a, rows_b, acc, sem_ga, sem_gb, sem_sa, sem_sb):
    cid = lax.axis_index("c")
    sid = lax.axis_index("s")
    wid = cid * NS + sid

    @pl.loop(0, K)
    def _(r):
        @pl.loop(0, D, step=16)
        def _(c):
            rows_a[r, pl.ds(c, 16)] = jnp.zeros((16,), jnp.float32)

    base = sid * RPT
    @pl.loop(0, RPT - (RPT % K), step=K)
    def _(j):
        pltpu.sync_copy(rows_a, acc.at[pl.ds(base + j, K)])
    pltpu.sync_copy(rows_a.at[pl.ds(0, RPT % K)],
                    acc.at[pl.ds(base + RPT - (RPT % K), RPT % K)])
    plsc.subcore_barrier()

    def run(g_hbm, ch_half, n_stages, rowbase):
        for h in range(n_stages):
            rb = rowbase + h * ch_half
            pltpu.sync_copy(srcm_hbm.at[pl.ds(rb, ch_half)],
                            src_v.at[pl.ds(0, ch_half)])
            pltpu.sync_copy(dstm_hbm.at[pl.ds(rb, ch_half)],
                            dst_v.at[pl.ds(0, ch_half)])
            pltpu.async_copy(g_hbm.at[src_v.at[0]], rows_a, sem_ga)
            pltpu.async_copy(g_hbm.at[src_v.at[1]], rows_b, sem_gb)

            @pl.loop(0, ch_half, step=2)
            def _(j):
                pltpu.make_async_copy(g_hbm.at[src_v.at[j]], rows_a,
                                      sem_ga).wait()
                pltpu.async_copy(rows_a, acc.at[dst_v.at[j]], sem_sa,
                                 add=True)
                pltpu.make_async_copy(g_hbm.at[src_v.at[j + 1]], rows_b,
                                      sem_gb).wait()
                pltpu.async_copy(rows_b, acc.at[dst_v.at[j + 1]], sem_sb,
                                 add=True)
                pltpu.make_async_copy(rows_a, acc.at[dst_v.at[j]],
                                      sem_sa).wait()

                @pl.when(j + 2 < ch_half)
                def _():
                    pltpu.async_copy(g_hbm.at[src_v.at[j + 2]], rows_a, sem_ga)
                pltpu.make_async_copy(rows_b, acc.at[dst_v.at[j + 1]],
                                      sem_sb).wait()

                @pl.when(j + 3 < ch_half)
                def _():
                    pltpu.async_copy(g_hbm.at[src_v.at[j + 3]], rows_b, sem_gb)

    @pl.when(cid == 0)
    def _():
        run(ga_hbm, ST0, CH0 // ST0, sid * CH0)

    @pl.when(cid == 1)
    def _():
        run(gb_hbm, ST1, CH1 // ST1, NS * CH0 + sid * CH1)

    plsc.subcore_barrier()
    pltpu.sync_copy(acc.at[pl.ds(base, RPT)], out_hbm.at[cid, pl.ds(base, RPT)])


@jax.jit
def _sc_scatter(ga, gb, srcm, dstm):
    f = pl.kernel(
        _scatter_body,
        out_type=jax.ShapeDtypeStruct((NC, N_SP, D), jnp.float32),
        mesh=_mesh,
        scratch_types=[
            pltpu.VMEM((ST0, K), jnp.int32),
            pltpu.VMEM((ST0, K), jnp.int32),
            pltpu.VMEM((K, D), jnp.float32),
            pltpu.VMEM((K, D), jnp.float32),
            pltpu.VMEM_SHARED((N_SP, D), jnp.float32),
            pltpu.SemaphoreType.DMA,
            pltpu.SemaphoreType.DMA,
            pltpu.SemaphoreType.DMA,
            pltpu.SemaphoreType.DMA,
        ],
    )
    return f(ga, gb, srcm, dstm)


def _tc1_body(x_ref, w1_ref, degp_ref, g1_ref, g1b_ref, dis_ref):
    deg = degp_ref[0, :N, 0:1] + degp_ref[1, :N, 0:1] + 1.0
    dis = lax.rsqrt(deg)
    h = jnp.dot(x_ref[...], w1_ref[...], preferred_element_type=jnp.float32)
    g1 = h * dis
    g1_ref[...] = g1
    g1b_ref[...] = g1
    dis_ref[...] = dis


def _tc2_body(g1_ref, aggp_ref, dis_ref, w2_ref, b1_ref, g2_ref, g2b_ref):
    agg = aggp_ref[0, :N, :] + aggp_ref[1, :N, :]
    dis = dis_ref[...]
    s = (agg + g1_ref[...]) * dis + b1_ref[...]
    h2 = jnp.maximum(s, 0.0)
    g2 = jnp.dot(h2, w2_ref[...], preferred_element_type=jnp.float32) * dis
    g2_ref[...] = g2
    g2b_ref[...] = g2


def _tc3_body(g2_ref, aggp_ref, dis_ref, b2_ref, ge_ref, out_ref):
    agg = aggp_ref[0, :N, :] + aggp_ref[1, :N, :]
    out = (agg + g2_ref[...]) * dis_ref[...] + b2_ref[...]
    out_ref[...] = out
    ge_ref[...] = jnp.mean(out, axis=0, keepdims=True)


@jax.jit
def _tc1(x, W1, degp):
    return pl.pallas_call(
        _tc1_body,
        out_shape=[jax.ShapeDtypeStruct((N, D), jnp.float32),
                   jax.ShapeDtypeStruct((N, D), jnp.float32),
                   jax.ShapeDtypeStruct((N, 1), jnp.float32)],
    )(x, W1, degp)


@jax.jit
def _tc2(g1, aggp, dis, W2, b1):
    return pl.pallas_call(
        _tc2_body,
        out_shape=[jax.ShapeDtypeStruct((N, D), jnp.float32),
                   jax.ShapeDtypeStruct((N, D), jnp.float32)],
    )(g1, aggp, dis, W2, b1)


@jax.jit
def _tc3(g2, aggp, dis, b2):
    return pl.pallas_call(
        _tc3_body,
        out_shape=[jax.ShapeDtypeStruct((1, D), jnp.float32),
                   jax.ShapeDtypeStruct((N, D), jnp.float32)],
    )(g2, aggp, dis, b2)


@jax.jit
def kernel(x, edge_index, W1, b1, W2, b2):
    ei = edge_index.astype(jnp.int32)
    pad = E_PAD - ei.shape[1]
    srcm = jnp.concatenate(
        [ei[0], jnp.zeros((pad,), jnp.int32)]).reshape(NW * CH, K)
    dstm = jnp.concatenate(
        [ei[1], jnp.full((pad,), PAD_ROW, jnp.int32)]).reshape(NW * CH, K)

    degp = _sc_deg(dstm)
    g1, g1b, dis = _tc1(x, W1, degp)
    aggp1 = _sc_scatter(g1, g1b, srcm, dstm)
    g2, g2b = _tc2(g1, aggp1, dis, W2, b1.reshape(1, D))
    aggp2 = _sc_scatter(g2, g2b, srcm, dstm)
    ge, out = _tc3(g2, aggp2, dis, b2.reshape(1, D))
    return (ge, out)
